# baseline (speedup 1.0000x reference)
import functools

import jax
import jax.numpy as jnp
from jax import lax
from jax.experimental import pallas as pl
from jax.experimental.pallas import tpu as pltpu
from jax.experimental.pallas import tpu_sc as plsc

_N = 10000
_E = 320000
_HID = 64
_HH = 32
_MU2P = 0.1
_BN_EPS = 1e-5
_NC = 2
_NS = 16
_NW = _NC * _NS
_CH = 128
_CHUNKS = 80
_EPW = _CHUNKS * _CH
_NP = 10112
_RPT = _NP // _NS
_DEGW = 16


def _sc_mesh():
    return plsc.VectorSubcoreMesh(core_axis_name="c", subcore_axis_name="s")


_SC_PARAMS = pltpu.CompilerParams(use_tc_tiling_on_sc=False)
_SC_PARAMS_NOLAYOUT = pltpu.CompilerParams(
    use_tc_tiling_on_sc=False, needs_layout_passes=False)


def _deg_partials(rowp, ones, zeros):
    @functools.partial(
        pl.kernel,
        out_type=jax.ShapeDtypeStruct((_NC, _NP, _DEGW), jnp.float32),
        mesh=_sc_mesh(),
        scratch_types=[
            pltpu.VMEM((_CHUNKS, _CH), jnp.int32),
            pltpu.VMEM((_CH, _DEGW), jnp.float32),
            pltpu.VMEM_SHARED((_NP, _DEGW), jnp.float32),
            pltpu.SemaphoreType.DMA,
        ],
        compiler_params=_SC_PARAMS,
    )
    def k(row_hbm, ones_hbm, zero_hbm, out_hbm, idx_v, ones_v, acc_sh, sem):
        c = lax.axis_index("c")
        s = lax.axis_index("s")
        wid = c * jnp.int32(_NS) + s
        sl = pl.ds(s * jnp.int32(_RPT), _RPT)
        pltpu.sync_copy(zero_hbm.at[sl], acc_sh.at[sl])
        pltpu.sync_copy(row_hbm.at[wid], idx_v)
        pltpu.sync_copy(ones_hbm, ones_v)
        plsc.subcore_barrier()

        @pl.loop(0, _CHUNKS, step=16)
        def _(g):
            @pl.loop(0, 16)
            def _(j):
                pltpu.async_copy(ones_v, acc_sh.at[idx_v.at[g + j]], sem,
                                 add=True)

            @pl.loop(0, 16)
            def _(j):
                pltpu.make_async_copy(
                    ones_v, acc_sh.at[idx_v.at[0]], sem).wait()

        plsc.subcore_barrier()
        pltpu.sync_copy(acc_sh.at[sl], out_hbm.at[c, sl])

    return k(rowp, ones, zeros)


def _spmv_partials(u, rowp, colp, zeros):
    @functools.partial(
        pl.kernel,
        out_type=jax.ShapeDtypeStruct((_NC, _NP, _HID), jnp.float32),
        mesh=_sc_mesh(),
        scratch_types=[
            pltpu.VMEM((_CHUNKS, _CH), jnp.int32),
            pltpu.VMEM((_CHUNKS, _CH), jnp.int32),
            pltpu.VMEM((8, _CH, _HH), jnp.bfloat16),
            pltpu.VMEM((8, _CH, _HH), jnp.float32),
            pltpu.VMEM_SHARED((_NP, _HH), jnp.bfloat16),
            pltpu.VMEM_SHARED((_NP, _HH), jnp.float32),
            pltpu.SemaphoreType.DMA((8,)),
            pltpu.SemaphoreType.DMA((8,)),
        ],
        compiler_params=_SC_PARAMS_NOLAYOUT,
    )
    def k(u_hbm, row_hbm, col_hbm, zero_hbm, out_hbm, row_v, col_v, rows_bf,
          rows_v, u_sh, acc_sh, gsem, ssem):
        c = lax.axis_index("c")
        s = lax.axis_index("s")
        wid = c * jnp.int32(_NS) + s
        sl = pl.ds(s * jnp.int32(_RPT), _RPT)
        pltpu.sync_copy(row_hbm.at[wid], row_v)
        pltpu.sync_copy(col_hbm.at[wid], col_v)

        def gather_start(i, b):
            pltpu.async_copy(u_sh.at[col_v.at[i]], rows_bf.at[b], gsem.at[b])

        def gather_wait(b):
            pltpu.make_async_copy(
                u_sh.at[col_v.at[0]], rows_bf.at[b], gsem.at[b]).wait()

        def convert(b):
            @pl.loop(0, _CH)
            def _(r):
                m = rows_bf[b, r, :]
                lo, hi = plsc.unpack(
                    m, format=plsc.PackFormat.INTERLEAVED,
                    preferred_element_type=jnp.float32)
                rows_v[b, r, pl.ds(0, 16)] = lo
                rows_v[b, r, pl.ds(16, 16)] = hi

        def scatter_start(i, b):
            pltpu.async_copy(rows_v.at[b], acc_sh.at[row_v.at[i]], ssem.at[b],
                             add=True)

        def scatter_wait(b):
            pltpu.make_async_copy(
                rows_v.at[b], acc_sh.at[row_v.at[0]], ssem.at[b]).wait()

        for h in range(2):
            hh = pl.ds(h * _HH, _HH)
            pltpu.sync_copy(zero_hbm, rows_v.at[0])
            base = s * jnp.int32(_RPT)
            for j in range(4):
                pltpu.sync_copy(
                    rows_v.at[0],
                    acc_sh.at[pl.ds(base + jnp.int32(j * _CH), _CH)])
            pltpu.sync_copy(rows_v.at[0, pl.ds(0, _RPT - 4 * _CH)],
                            acc_sh.at[pl.ds(base + jnp.int32(4 * _CH),
                                            _RPT - 4 * _CH)])
            pltpu.sync_copy(u_hbm.at[sl, hh], u_sh.at[sl])
            plsc.subcore_barrier()

            for b in range(4):
                gather_start(b, b)

            @pl.loop(0, _CHUNKS, step=8)
            def _(g):
                for b in range(8):
                    i = g + jnp.int32(b)
                    b2 = (b + 4) % 8
                    gather_wait(b)
                    convert(b)
                    scatter_start(i, b)

                    @pl.when(i >= 4)
                    def _():
                        scatter_wait(b2)

                    @pl.when(i + 4 < _CHUNKS)
                    def _():
                        gather_start(i + 4, b2)

            for b in range(4, 8):
                scatter_wait(b)
            plsc.subcore_barrier()
            pltpu.sync_copy(acc_sh.at[sl], out_hbm.at[c, sl, hh])

    return k(u, rowp, colp, zeros)


def _to_sc_bf16(u):
    u4 = u.reshape(_NP, 2, 2, 16)
    return u4.transpose(0, 1, 3, 2).reshape(_NP, _HID).astype(jnp.bfloat16)


def _prep_h(x, W1, b1, gamma, beta, mean, var):
    def body(x_ref, w_ref, b_ref, g_ref, be_ref, m_ref, v_ref, o_ref):
        hm = jnp.dot(x_ref[...], w_ref[...], preferred_element_type=jnp.float32)
        inv = g_ref[...] * lax.rsqrt(v_ref[...] + _BN_EPS)
        o_ref[pl.ds(0, _N), :] = jnp.maximum(
            (hm + b_ref[...] - m_ref[...]) * inv + be_ref[...], 0.0)
        o_ref[pl.ds(_N, _NP - _N), :] = jnp.zeros(
            (_NP - _N, _HID), jnp.float32)

    return pl.pallas_call(
        body, out_shape=jax.ShapeDtypeStruct((_NP, _HID), jnp.float32)
    )(x, W1, b1, gamma, beta, mean, var)


def _node_coeffs(d_ref):
    deg = d_ref[0, :, 0:1] + d_ref[1, :, 0:1]
    pos = deg > 0.0
    dis = jnp.where(pos, lax.rsqrt(deg), 0.0)
    alpha = 1.0 / (jnp.where(pos, 1.0, 0.0) + _MU2P)
    return dis, alpha


def _make_u0(degp, h):
    def body(d_ref, h_ref, o_ref):
        dis, _ = _node_coeffs(d_ref)
        o_ref[...] = dis * h_ref[...]

    return pl.pallas_call(
        body, out_shape=jax.ShapeDtypeStruct((_NP, _HID), jnp.float32)
    )(degp, h)


def _combine_u(degp, s, h):
    def body(d_ref, s_ref, h_ref, o_ref):
        dis, alpha = _node_coeffs(d_ref)
        f = alpha * (dis * (s_ref[0] + s_ref[1])) + (_MU2P * alpha) * h_ref[...]
        o_ref[...] = dis * f

    return pl.pallas_call(
        body, out_shape=jax.ShapeDtypeStruct((_NP, _HID), jnp.float32)
    )(degp, s, h)


def _final(degp, s, h, W2, b2):
    def body(d_ref, s_ref, h_ref, w_ref, b_ref, o_ref):
        dis, alpha = _node_coeffs(d_ref)
        f = alpha * (dis * (s_ref[0] + s_ref[1])) + (_MU2P * alpha) * h_ref[...]
        o = jnp.dot(f, w_ref[...], preferred_element_type=jnp.float32) + b_ref[...]
        m = jnp.max(o, axis=1, keepdims=True)
        y = o - m
        o_ref[...] = y - jnp.log(jnp.sum(jnp.exp(y), axis=1, keepdims=True))

    return pl.pallas_call(
        body, out_shape=jax.ShapeDtypeStruct((_NP, _HID), jnp.float32)
    )(degp, s, h, W2, b2)


def kernel(x, edge_index, W1, b1, gamma, beta, running_mean, running_var, W2, b2):
    with jax.enable_x64(False):
        return _kernel32(x, edge_index, W1, b1, gamma, beta, running_mean,
                         running_var, W2, b2)


def _kernel32(x, edge_index, W1, b1, gamma, beta, running_mean, running_var, W2, b2):
    x = x.astype(jnp.float32)
    row = edge_index[0].astype(jnp.int32)
    col = edge_index[1].astype(jnp.int32)
    pad = _NW * _EPW - _E
    rowp = jnp.concatenate([row, jnp.full((pad,), _N, jnp.int32)]).reshape(
        _NW, _CHUNKS, _CH)
    colp = jnp.concatenate([col, jnp.full((pad,), _N, jnp.int32)]).reshape(
        _NW, _CHUNKS, _CH)
    ones16 = jnp.ones((_CH, _DEGW), jnp.float32)
    z16 = jnp.zeros((_NP, _DEGW), jnp.float32)
    zblk = jnp.zeros((_CH, _HH), jnp.float32)

    h = _prep_h(x, W1.astype(jnp.float32), b1, gamma, beta,
                running_mean, running_var)
    degp = _deg_partials(rowp, ones16, z16)
    u = _make_u0(degp, h)
    s = _spmv_partials(_to_sc_bf16(u), rowp, colp, zblk)
    u = _combine_u(degp, s, h)
    s = _spmv_partials(_to_sc_bf16(u), rowp, colp, zblk)
    out = _final(degp, s, h, W2.astype(jnp.float32), b2)
    return out[:_N]


# speedup vs baseline: 1.3344x; 1.3344x over previous
import functools

import jax
import jax.numpy as jnp
from jax import lax
from jax.experimental import pallas as pl
from jax.experimental.pallas import tpu as pltpu
from jax.experimental.pallas import tpu_sc as plsc

_N = 10000
_E = 320000
_HID = 64
_HH = 32
_MU2P = 0.1
_BN_EPS = 1e-5
_NC = 2
_NS = 16
_NW = _NC * _NS
_CH = 128
_CHUNKS = 80
_EPW = _CHUNKS * _CH
_NP = 10112
_RPT = _NP // _NS
_DEGW = 16


def _sc_mesh():
    return plsc.VectorSubcoreMesh(core_axis_name="c", subcore_axis_name="s")


_SC_PARAMS = pltpu.CompilerParams(use_tc_tiling_on_sc=False)
_SC_PARAMS_NOLAYOUT = pltpu.CompilerParams(
    use_tc_tiling_on_sc=False, needs_layout_passes=False)


def _deg_partials(rowp, ones, zeros):
    @functools.partial(
        pl.kernel,
        out_type=jax.ShapeDtypeStruct((_NC, _NP, _DEGW), jnp.float32),
        mesh=_sc_mesh(),
        scratch_types=[
            pltpu.VMEM((_CHUNKS, _CH), jnp.int32),
            pltpu.VMEM((_CH, _DEGW), jnp.float32),
            pltpu.VMEM_SHARED((_NP, _DEGW), jnp.float32),
            pltpu.SemaphoreType.DMA,
        ],
        compiler_params=_SC_PARAMS,
    )
    def k(row_hbm, ones_hbm, zero_hbm, out_hbm, idx_v, ones_v, acc_sh, sem):
        c = lax.axis_index("c")
        s = lax.axis_index("s")
        wid = c * jnp.int32(_NS) + s
        sl = pl.ds(s * jnp.int32(_RPT), _RPT)
        pltpu.sync_copy(zero_hbm.at[sl], acc_sh.at[sl])
        pltpu.sync_copy(row_hbm.at[wid], idx_v)
        pltpu.sync_copy(ones_hbm, ones_v)
        plsc.subcore_barrier()

        @pl.loop(0, _CHUNKS, step=16)
        def _(g):
            @pl.loop(0, 16)
            def _(j):
                pltpu.async_copy(ones_v, acc_sh.at[idx_v.at[g + j]], sem,
                                 add=True)

            @pl.loop(0, 16)
            def _(j):
                pltpu.make_async_copy(
                    ones_v, acc_sh.at[idx_v.at[0]], sem).wait()

        plsc.subcore_barrier()
        pltpu.sync_copy(acc_sh.at[sl], out_hbm.at[c, sl])

    return k(rowp, ones, zeros)


def _spmv_partials(u, rowp, colp, zeros):
    @functools.partial(
        pl.kernel,
        out_type=jax.ShapeDtypeStruct((_NC, _NP, _HID), jnp.float32),
        mesh=_sc_mesh(),
        scratch_types=[
            pltpu.VMEM((_CHUNKS, _CH), jnp.int32),
            pltpu.VMEM((_CHUNKS, _CH), jnp.int32),
            pltpu.VMEM((8, _CH, _HH), jnp.float32),
            pltpu.VMEM_SHARED((_NP, _HH), jnp.float32),
            pltpu.VMEM_SHARED((_NP, _HH), jnp.float32),
            pltpu.SemaphoreType.DMA((8,)),
            pltpu.SemaphoreType.DMA((8,)),
        ],
        compiler_params=_SC_PARAMS,
    )
    def k(u_hbm, row_hbm, col_hbm, zero_hbm, out_hbm, row_v, col_v,
          rows_v, u_sh, acc_sh, gsem, ssem):
        c = lax.axis_index("c")
        s = lax.axis_index("s")
        wid = c * jnp.int32(_NS) + s
        sl = pl.ds(s * jnp.int32(_RPT), _RPT)
        pltpu.sync_copy(row_hbm.at[wid], row_v)
        pltpu.sync_copy(col_hbm.at[wid], col_v)

        def gather_start(i, b):
            pltpu.async_copy(u_sh.at[col_v.at[i]], rows_v.at[b], gsem.at[b])

        def gather_wait(b):
            pltpu.make_async_copy(
                u_sh.at[col_v.at[0]], rows_v.at[b], gsem.at[b]).wait()

        def scatter_start(i, b):
            pltpu.async_copy(rows_v.at[b], acc_sh.at[row_v.at[i]], ssem.at[b],
                             add=True)

        def scatter_wait(b):
            pltpu.make_async_copy(
                rows_v.at[b], acc_sh.at[row_v.at[0]], ssem.at[b]).wait()

        for h in range(2):
            hh = pl.ds(h * _HH, _HH)
            pltpu.sync_copy(zero_hbm, rows_v.at[0])
            base = s * jnp.int32(_RPT)
            for j in range(4):
                pltpu.sync_copy(
                    rows_v.at[0],
                    acc_sh.at[pl.ds(base + jnp.int32(j * _CH), _CH)])
            pltpu.sync_copy(rows_v.at[0, pl.ds(0, _RPT - 4 * _CH)],
                            acc_sh.at[pl.ds(base + jnp.int32(4 * _CH),
                                            _RPT - 4 * _CH)])
            pltpu.sync_copy(u_hbm.at[sl, hh], u_sh.at[sl])
            plsc.subcore_barrier()

            for b in range(4):
                gather_start(b, b)

            @pl.loop(0, _CHUNKS, step=8)
            def _(g):
                for b in range(8):
                    i = g + jnp.int32(b)
                    b2 = (b + 4) % 8
                    gather_wait(b)
                    scatter_start(i, b)

                    @pl.when(i >= 4)
                    def _():
                        scatter_wait(b2)

                    @pl.when(i + 4 < _CHUNKS)
                    def _():
                        gather_start(i + 4, b2)

            for b in range(4, 8):
                scatter_wait(b)
            plsc.subcore_barrier()
            pltpu.sync_copy(acc_sh.at[sl], out_hbm.at[c, sl, hh])

    return k(u, rowp, colp, zeros)


def _to_sc_bf16(u):
    u4 = u.reshape(_NP, 2, 2, 16)
    return u4.transpose(0, 1, 3, 2).reshape(_NP, _HID).astype(jnp.bfloat16)


def _prep_h(x, W1, b1, gamma, beta, mean, var):
    def body(x_ref, w_ref, b_ref, g_ref, be_ref, m_ref, v_ref, o_ref):
        hm = jnp.dot(x_ref[...], w_ref[...], preferred_element_type=jnp.float32)
        inv = g_ref[...] * lax.rsqrt(v_ref[...] + _BN_EPS)
        o_ref[pl.ds(0, _N), :] = jnp.maximum(
            (hm + b_ref[...] - m_ref[...]) * inv + be_ref[...], 0.0)
        o_ref[pl.ds(_N, _NP - _N), :] = jnp.zeros(
            (_NP - _N, _HID), jnp.float32)

    return pl.pallas_call(
        body, out_shape=jax.ShapeDtypeStruct((_NP, _HID), jnp.float32)
    )(x, W1, b1, gamma, beta, mean, var)


def _node_coeffs(d_ref):
    deg = d_ref[0, :, 0:1] + d_ref[1, :, 0:1]
    pos = deg > 0.0
    dis = jnp.where(pos, lax.rsqrt(deg), 0.0)
    alpha = 1.0 / (jnp.where(pos, 1.0, 0.0) + _MU2P)
    return dis, alpha


def _make_u0(degp, h):
    def body(d_ref, h_ref, o_ref):
        dis, _ = _node_coeffs(d_ref)
        o_ref[...] = dis * h_ref[...]

    return pl.pallas_call(
        body, out_shape=jax.ShapeDtypeStruct((_NP, _HID), jnp.float32)
    )(degp, h)


def _combine_u(degp, s, h):
    def body(d_ref, s_ref, h_ref, o_ref):
        dis, alpha = _node_coeffs(d_ref)
        f = alpha * (dis * (s_ref[0] + s_ref[1])) + (_MU2P * alpha) * h_ref[...]
        o_ref[...] = dis * f

    return pl.pallas_call(
        body, out_shape=jax.ShapeDtypeStruct((_NP, _HID), jnp.float32)
    )(degp, s, h)


def _final(degp, s, h, W2, b2):
    def body(d_ref, s_ref, h_ref, w_ref, b_ref, o_ref):
        dis, alpha = _node_coeffs(d_ref)
        f = alpha * (dis * (s_ref[0] + s_ref[1])) + (_MU2P * alpha) * h_ref[...]
        o = jnp.dot(f, w_ref[...], preferred_element_type=jnp.float32) + b_ref[...]
        m = jnp.max(o, axis=1, keepdims=True)
        y = o - m
        o_ref[...] = y - jnp.log(jnp.sum(jnp.exp(y), axis=1, keepdims=True))

    return pl.pallas_call(
        body, out_shape=jax.ShapeDtypeStruct((_NP, _HID), jnp.float32)
    )(degp, s, h, W2, b2)


def kernel(x, edge_index, W1, b1, gamma, beta, running_mean, running_var, W2, b2):
    with jax.enable_x64(False):
        return _kernel32(x, edge_index, W1, b1, gamma, beta, running_mean,
                         running_var, W2, b2)


def _kernel32(x, edge_index, W1, b1, gamma, beta, running_mean, running_var, W2, b2):
    x = x.astype(jnp.float32)
    row = edge_index[0].astype(jnp.int32)
    col = edge_index[1].astype(jnp.int32)
    pad = _NW * _EPW - _E
    rowp = jnp.concatenate([row, jnp.full((pad,), _N, jnp.int32)]).reshape(
        _NW, _CHUNKS, _CH)
    colp = jnp.concatenate([col, jnp.full((pad,), _N, jnp.int32)]).reshape(
        _NW, _CHUNKS, _CH)
    ones16 = jnp.ones((_CH, _DEGW), jnp.float32)
    z16 = jnp.zeros((_NP, _DEGW), jnp.float32)
    zblk = jnp.zeros((_CH, _HH), jnp.float32)

    h = _prep_h(x, W1.astype(jnp.float32), b1, gamma, beta,
                running_mean, running_var)
    degp = _deg_partials(rowp, ones16, z16)
    u = _make_u0(degp, h)
    s = _spmv_partials(u, rowp, colp, zblk)
    u = _combine_u(degp, s, h)
    s = _spmv_partials(u, rowp, colp, zblk)
    out = _final(degp, s, h, W2.astype(jnp.float32), b2)
    return out[:_N]


# speedup vs baseline: 1.3501x; 1.0118x over previous
import functools

import jax
import jax.numpy as jnp
from jax import lax
from jax.experimental import pallas as pl
from jax.experimental.pallas import tpu as pltpu
from jax.experimental.pallas import tpu_sc as plsc

_N = 10000
_E = 320000
_HID = 64
_HH = 32
_MU2P = 0.1
_BN_EPS = 1e-5
_NC = 2
_NS = 16
_NW = _NC * _NS
_CH = 128
_CHUNKS = 80
_EPW = _CHUNKS * _CH
_NP = 10112
_RPT = _NP // _NS
_DEGW = 16


def _sc_mesh():
    return plsc.VectorSubcoreMesh(core_axis_name="c", subcore_axis_name="s")


_SC_PARAMS = pltpu.CompilerParams(use_tc_tiling_on_sc=False)
_SC_PARAMS_NOLAYOUT = pltpu.CompilerParams(
    use_tc_tiling_on_sc=False, needs_layout_passes=False)


def _deg_partials(rowp, ones, zeros):
    @functools.partial(
        pl.kernel,
        out_type=jax.ShapeDtypeStruct((_NC, _NP, _DEGW), jnp.float32),
        mesh=_sc_mesh(),
        scratch_types=[
            pltpu.VMEM((_CHUNKS, _CH), jnp.int32),
            pltpu.VMEM((_CH, _DEGW), jnp.float32),
            pltpu.VMEM_SHARED((_NP, _DEGW), jnp.float32),
            pltpu.SemaphoreType.DMA,
        ],
        compiler_params=_SC_PARAMS,
    )
    def k(row_hbm, ones_hbm, zero_hbm, out_hbm, idx_v, ones_v, acc_sh, sem):
        c = lax.axis_index("c")
        s = lax.axis_index("s")
        wid = c * jnp.int32(_NS) + s
        sl = pl.ds(s * jnp.int32(_RPT), _RPT)
        pltpu.sync_copy(zero_hbm.at[sl], acc_sh.at[sl])
        pltpu.sync_copy(row_hbm.at[wid], idx_v)
        pltpu.sync_copy(ones_hbm, ones_v)
        plsc.subcore_barrier()

        @pl.loop(0, _CHUNKS, step=16)
        def _(g):
            @pl.loop(0, 16)
            def _(j):
                pltpu.async_copy(ones_v, acc_sh.at[idx_v.at[g + j]], sem,
                                 add=True)

            @pl.loop(0, 16)
            def _(j):
                pltpu.make_async_copy(
                    ones_v, acc_sh.at[idx_v.at[0]], sem).wait()

        plsc.subcore_barrier()
        pltpu.sync_copy(acc_sh.at[sl], out_hbm.at[c, sl])

    return k(rowp, ones, zeros)


def _spmv_partials(u, rowp, colp, zeros):
    @functools.partial(
        pl.kernel,
        out_type=jax.ShapeDtypeStruct((_NC, _NP, _HID), jnp.float32),
        mesh=_sc_mesh(),
        scratch_types=[
            pltpu.VMEM((_CHUNKS, _CH), jnp.int32),
            pltpu.VMEM((_CHUNKS, _CH), jnp.int32),
            pltpu.VMEM((8, _CH, _HH), jnp.float32),
            pltpu.VMEM_SHARED((_NP, _HH), jnp.float32),
            pltpu.VMEM_SHARED((_NP, _HH), jnp.float32),
            pltpu.SemaphoreType.DMA((8,)),
            pltpu.SemaphoreType.DMA((8,)),
        ],
        compiler_params=_SC_PARAMS,
    )
    def k(u_hbm, row_hbm, col_hbm, zero_hbm, out_hbm, row_v, col_v,
          rows_v, u_sh, acc_sh, gsem, ssem):
        c = lax.axis_index("c")
        s = lax.axis_index("s")
        wid = c * jnp.int32(_NS) + s
        sl = pl.ds(s * jnp.int32(_RPT), _RPT)
        pltpu.sync_copy(row_hbm.at[wid], row_v)
        pltpu.sync_copy(col_hbm.at[wid], col_v)

        def gather_start(i, b):
            pltpu.async_copy(u_sh.at[col_v.at[i]], rows_v.at[b], gsem.at[b])

        def gather_wait(b):
            pltpu.make_async_copy(
                u_sh.at[col_v.at[0]], rows_v.at[b], gsem.at[b]).wait()

        def scatter_start(i, b):
            pltpu.async_copy(rows_v.at[b], acc_sh.at[row_v.at[i]], ssem.at[b],
                             add=True)

        def scatter_wait(b):
            pltpu.make_async_copy(
                rows_v.at[b], acc_sh.at[row_v.at[0]], ssem.at[b]).wait()

        for h in range(2):
            hh = pl.ds(h * _HH, _HH)
            pltpu.sync_copy(zero_hbm, rows_v.at[0])
            base = s * jnp.int32(_RPT)
            for j in range(4):
                pltpu.sync_copy(
                    rows_v.at[0],
                    acc_sh.at[pl.ds(base + jnp.int32(j * _CH), _CH)])
            pltpu.sync_copy(rows_v.at[0, pl.ds(0, _RPT - 4 * _CH)],
                            acc_sh.at[pl.ds(base + jnp.int32(4 * _CH),
                                            _RPT - 4 * _CH)])
            pltpu.sync_copy(u_hbm.at[sl, hh], u_sh.at[sl])
            plsc.subcore_barrier()

            for b in range(4):
                gather_start(b, b)

            @pl.loop(0, _CHUNKS, step=8)
            def _(g):
                for b in range(8):
                    i = g + jnp.int32(b)
                    b2 = (b + 4) % 8
                    gather_wait(b)
                    scatter_start(i, b)

                    @pl.when(i >= 4)
                    def _():
                        scatter_wait(b2)

                    @pl.when(i + 4 < _CHUNKS)
                    def _():
                        gather_start(i + 4, b2)

            for b in range(4, 8):
                scatter_wait(b)
            plsc.subcore_barrier()
            pltpu.sync_copy(acc_sh.at[sl], out_hbm.at[c, sl, hh])

    return k(u, rowp, colp, zeros)


def _to_sc_bf16(u):
    u4 = u.reshape(_NP, 2, 2, 16)
    return u4.transpose(0, 1, 3, 2).reshape(_NP, _HID).astype(jnp.bfloat16)


def _prep_h(x, W1, b1, gamma, beta, mean, var):
    def body(x_ref, w_ref, b_ref, g_ref, be_ref, m_ref, v_ref, o_ref):
        hm = jnp.dot(x_ref[...], w_ref[...], preferred_element_type=jnp.float32)
        inv = g_ref[...] * lax.rsqrt(v_ref[...] + _BN_EPS)
        o_ref[pl.ds(0, _N), :] = jnp.maximum(
            (hm + b_ref[...] - m_ref[...]) * inv + be_ref[...], 0.0)
        o_ref[pl.ds(_N, _NP - _N), :] = jnp.zeros(
            (_NP - _N, _HID), jnp.float32)

    return pl.pallas_call(
        body, out_shape=jax.ShapeDtypeStruct((_NP, _HID), jnp.float32)
    )(x, W1, b1, gamma, beta, mean, var)


def _node_coeffs(d_ref):
    deg = d_ref[0, :, 0:1] + d_ref[1, :, 0:1]
    pos = deg > 0.0
    dis = jnp.where(pos, lax.rsqrt(deg), 0.0)
    alpha = 1.0 / (jnp.where(pos, 1.0, 0.0) + _MU2P)
    return dis, alpha


_GB = 8
_BR = _NP // _GB

_D_SPEC = pl.BlockSpec((2, _BR, _DEGW), lambda i: (0, i, 0))
_S_SPEC = pl.BlockSpec((_NC, _BR, _HID), lambda i: (0, i, 0))
_H_SPEC = pl.BlockSpec((_BR, _HID), lambda i: (i, 0))


def _make_u0(degp, h):
    def body(d_ref, h_ref, o_ref):
        dis, _ = _node_coeffs(d_ref)
        o_ref[...] = dis * h_ref[...]

    return pl.pallas_call(
        body, grid=(_GB,),
        in_specs=[_D_SPEC, _H_SPEC], out_specs=_H_SPEC,
        out_shape=jax.ShapeDtypeStruct((_NP, _HID), jnp.float32),
    )(degp, h)


def _combine_u(degp, s, h):
    def body(d_ref, s_ref, h_ref, o_ref):
        dis, alpha = _node_coeffs(d_ref)
        f = alpha * (dis * (s_ref[0] + s_ref[1])) + (_MU2P * alpha) * h_ref[...]
        o_ref[...] = dis * f

    return pl.pallas_call(
        body, grid=(_GB,),
        in_specs=[_D_SPEC, _S_SPEC, _H_SPEC], out_specs=_H_SPEC,
        out_shape=jax.ShapeDtypeStruct((_NP, _HID), jnp.float32),
    )(degp, s, h)


def _final(degp, s, h, W2, b2):
    def body(d_ref, s_ref, h_ref, w_ref, b_ref, o_ref):
        dis, alpha = _node_coeffs(d_ref)
        f = alpha * (dis * (s_ref[0] + s_ref[1])) + (_MU2P * alpha) * h_ref[...]
        o = jnp.dot(f, w_ref[...], preferred_element_type=jnp.float32) + b_ref[...]
        m = jnp.max(o, axis=1, keepdims=True)
        y = o - m
        o_ref[...] = y - jnp.log(jnp.sum(jnp.exp(y), axis=1, keepdims=True))

    return pl.pallas_call(
        body, grid=(_GB,),
        in_specs=[_D_SPEC, _S_SPEC, _H_SPEC,
                  pl.BlockSpec((_HID, _HID), lambda i: (0, 0)),
                  pl.BlockSpec((_HID,), lambda i: (0,))],
        out_specs=_H_SPEC,
        out_shape=jax.ShapeDtypeStruct((_NP, _HID), jnp.float32),
    )(degp, s, h, W2, b2)


def kernel(x, edge_index, W1, b1, gamma, beta, running_mean, running_var, W2, b2):
    with jax.enable_x64(False):
        return _kernel32(x, edge_index, W1, b1, gamma, beta, running_mean,
                         running_var, W2, b2)


def _kernel32(x, edge_index, W1, b1, gamma, beta, running_mean, running_var, W2, b2):
    x = x.astype(jnp.float32)
    row = edge_index[0].astype(jnp.int32)
    col = edge_index[1].astype(jnp.int32)
    pad = _NW * _EPW - _E
    rowp = jnp.concatenate([row, jnp.full((pad,), _N, jnp.int32)]).reshape(
        _NW, _CHUNKS, _CH)
    colp = jnp.concatenate([col, jnp.full((pad,), _N, jnp.int32)]).reshape(
        _NW, _CHUNKS, _CH)
    ones16 = jnp.ones((_CH, _DEGW), jnp.float32)
    z16 = jnp.zeros((_NP, _DEGW), jnp.float32)
    zblk = jnp.zeros((_CH, _HH), jnp.float32)

    h = _prep_h(x, W1.astype(jnp.float32), b1, gamma, beta,
                running_mean, running_var)
    degp = _deg_partials(rowp, ones16, z16)
    u = _make_u0(degp, h)
    s = _spmv_partials(u, rowp, colp, zblk)
    u = _combine_u(degp, s, h)
    s = _spmv_partials(u, rowp, colp, zblk)
    out = _final(degp, s, h, W2.astype(jnp.float32), b2)
    return out[:_N]


# speedup vs baseline: 1.3508x; 1.0005x over previous
import functools

import jax
import jax.numpy as jnp
from jax import lax
from jax.experimental import pallas as pl
from jax.experimental.pallas import tpu as pltpu
from jax.experimental.pallas import tpu_sc as plsc

_N = 10000
_E = 320000
_HID = 64
_HH = 32
_MU2P = 0.1
_BN_EPS = 1e-5
_NC = 2
_NS = 16
_NW = _NC * _NS
_CH = 128
_CHUNKS = 80
_EPW = _CHUNKS * _CH
_NP = 10112
_RPT = _NP // _NS
_DEGW = 16


def _sc_mesh():
    return plsc.VectorSubcoreMesh(core_axis_name="c", subcore_axis_name="s")


_SC_PARAMS = pltpu.CompilerParams(use_tc_tiling_on_sc=False)


def _deg_partials(rowp, ones, zeros):
    @functools.partial(
        pl.kernel,
        out_type=jax.ShapeDtypeStruct((_NC, _NP, _DEGW), jnp.float32),
        mesh=_sc_mesh(),
        scratch_types=[
            pltpu.VMEM((_CHUNKS, _CH), jnp.int32),
            pltpu.VMEM((_CH, _DEGW), jnp.float32),
            pltpu.VMEM_SHARED((_NP, _DEGW), jnp.float32),
            pltpu.SemaphoreType.DMA,
        ],
        compiler_params=_SC_PARAMS,
    )
    def k(row_hbm, ones_hbm, zero_hbm, out_hbm, idx_v, ones_v, acc_sh, sem):
        c = lax.axis_index("c")
        s = lax.axis_index("s")
        wid = c * jnp.int32(_NS) + s
        sl = pl.ds(s * jnp.int32(_RPT), _RPT)
        pltpu.sync_copy(zero_hbm.at[sl], acc_sh.at[sl])
        pltpu.sync_copy(row_hbm.at[wid], idx_v)
        pltpu.sync_copy(ones_hbm, ones_v)
        plsc.subcore_barrier()

        @pl.loop(0, _CHUNKS, step=16)
        def _(g):
            @pl.loop(0, 16)
            def _(j):
                pltpu.async_copy(ones_v, acc_sh.at[idx_v.at[g + j]], sem,
                                 add=True)

            @pl.loop(0, 16)
            def _(j):
                pltpu.make_async_copy(
                    ones_v, acc_sh.at[idx_v.at[0]], sem).wait()

        plsc.subcore_barrier()
        pltpu.sync_copy(acc_sh.at[sl], out_hbm.at[c, sl])

    return k(rowp, ones, zeros)


def _spmv_partials(u, rowp, colp, zeros):
    @functools.partial(
        pl.kernel,
        out_type=jax.ShapeDtypeStruct((_NC, _NP, _HID), jnp.float32),
        mesh=_sc_mesh(),
        scratch_types=[
            pltpu.VMEM((_CHUNKS, _CH), jnp.int32),
            pltpu.VMEM((_CHUNKS, _CH), jnp.int32),
            pltpu.VMEM((8, _CH, _HH), jnp.float32),
            pltpu.VMEM_SHARED((_NP, _HH), jnp.float32),
            pltpu.VMEM_SHARED((_NP, _HH), jnp.float32),
            pltpu.SemaphoreType.DMA((8,)),
            pltpu.SemaphoreType.DMA((8,)),
        ],
        compiler_params=_SC_PARAMS,
    )
    def k(u_hbm, row_hbm, col_hbm, zero_hbm, out_hbm, row_v, col_v,
          rows_v, u_sh, acc_sh, gsem, ssem):
        c = lax.axis_index("c")
        s = lax.axis_index("s")
        wid = c * jnp.int32(_NS) + s
        sl = pl.ds(s * jnp.int32(_RPT), _RPT)
        pltpu.sync_copy(row_hbm.at[wid], row_v)
        pltpu.sync_copy(col_hbm.at[wid], col_v)

        def gather_start(i, b):
            pltpu.async_copy(u_sh.at[col_v.at[i]], rows_v.at[b], gsem.at[b])

        def gather_wait(b):
            pltpu.make_async_copy(
                u_sh.at[col_v.at[0]], rows_v.at[b], gsem.at[b]).wait()

        def scatter_start(i, b):
            pltpu.async_copy(rows_v.at[b], acc_sh.at[row_v.at[i]], ssem.at[b],
                             add=True)

        def scatter_wait(b):
            pltpu.make_async_copy(
                rows_v.at[b], acc_sh.at[row_v.at[0]], ssem.at[b]).wait()

        for h in range(2):
            hh = pl.ds(h * _HH, _HH)
            pltpu.sync_copy(zero_hbm, rows_v.at[0])
            base = s * jnp.int32(_RPT)
            for j in range(4):
                pltpu.sync_copy(
                    rows_v.at[0],
                    acc_sh.at[pl.ds(base + jnp.int32(j * _CH), _CH)])
            pltpu.sync_copy(rows_v.at[0, pl.ds(0, _RPT - 4 * _CH)],
                            acc_sh.at[pl.ds(base + jnp.int32(4 * _CH),
                                            _RPT - 4 * _CH)])
            pltpu.sync_copy(u_hbm.at[sl, hh], u_sh.at[sl])
            plsc.subcore_barrier()

            for b in range(4):
                gather_start(b, b)

            @pl.loop(0, _CHUNKS, step=8)
            def _(g):
                for b in range(8):
                    i = g + jnp.int32(b)
                    b2 = (b + 4) % 8
                    gather_wait(b)
                    scatter_start(i, b)

                    @pl.when(i >= 4)
                    def _():
                        scatter_wait(b2)

                    @pl.when(i + 4 < _CHUNKS)
                    def _():
                        gather_start(i + 4, b2)

            for b in range(4, 8):
                scatter_wait(b)
            plsc.subcore_barrier()
            pltpu.sync_copy(acc_sh.at[sl], out_hbm.at[c, sl, hh])

    return k(u, rowp, colp, zeros)


def _prep_h(x, W1, b1, gamma, beta, mean, var):
    def body(x_ref, w_ref, b_ref, g_ref, be_ref, m_ref, v_ref, o_ref):
        hm = jnp.dot(x_ref[...], w_ref[...], preferred_element_type=jnp.float32)
        inv = g_ref[...] * lax.rsqrt(v_ref[...] + _BN_EPS)
        o_ref[pl.ds(0, _N), :] = jnp.maximum(
            (hm + b_ref[...] - m_ref[...]) * inv + be_ref[...], 0.0)
        o_ref[pl.ds(_N, _NP - _N), :] = jnp.zeros(
            (_NP - _N, _HID), jnp.float32)

    return pl.pallas_call(
        body, out_shape=jax.ShapeDtypeStruct((_NP, _HID), jnp.float32)
    )(x, W1, b1, gamma, beta, mean, var)


def _node_coeffs(d_ref):
    deg = d_ref[0, :, 0:1] + d_ref[1, :, 0:1]
    pos = deg > 0.0
    dis = jnp.where(pos, lax.rsqrt(deg), 0.0)
    alpha = 1.0 / (jnp.where(pos, 1.0, 0.0) + _MU2P)
    return dis, alpha


_GB = 8
_BR = _NP // _GB

_D_SPEC = pl.BlockSpec((2, _BR, _DEGW), lambda i: (0, i, 0))
_S_SPEC = pl.BlockSpec((_NC, _BR, _HID), lambda i: (0, i, 0))
_H_SPEC = pl.BlockSpec((_BR, _HID), lambda i: (i, 0))


def _make_u0(degp, h):
    def body(d_ref, h_ref, o_ref):
        dis, _ = _node_coeffs(d_ref)
        o_ref[...] = dis * h_ref[...]

    return pl.pallas_call(
        body, grid=(_GB,),
        in_specs=[_D_SPEC, _H_SPEC], out_specs=_H_SPEC,
        out_shape=jax.ShapeDtypeStruct((_NP, _HID), jnp.float32),
    )(degp, h)


def _combine_u(degp, s, h):
    def body(d_ref, s_ref, h_ref, o_ref):
        dis, alpha = _node_coeffs(d_ref)
        f = alpha * (dis * (s_ref[0] + s_ref[1])) + (_MU2P * alpha) * h_ref[...]
        o_ref[...] = dis * f

    return pl.pallas_call(
        body, grid=(_GB,),
        in_specs=[_D_SPEC, _S_SPEC, _H_SPEC], out_specs=_H_SPEC,
        out_shape=jax.ShapeDtypeStruct((_NP, _HID), jnp.float32),
    )(degp, s, h)


def _final(degp, s, h, W2, b2):
    def body(d_ref, s_ref, h_ref, w_ref, b_ref, o_ref):
        dis, alpha = _node_coeffs(d_ref)
        f = alpha * (dis * (s_ref[0] + s_ref[1])) + (_MU2P * alpha) * h_ref[...]
        o = jnp.dot(f, w_ref[...], preferred_element_type=jnp.float32) + b_ref[...]
        m = jnp.max(o, axis=1, keepdims=True)
        y = o - m
        o_ref[...] = y - jnp.log(jnp.sum(jnp.exp(y), axis=1, keepdims=True))

    return pl.pallas_call(
        body, grid=(_GB,),
        in_specs=[_D_SPEC, _S_SPEC, _H_SPEC,
                  pl.BlockSpec((_HID, _HID), lambda i: (0, 0)),
                  pl.BlockSpec((_HID,), lambda i: (0,))],
        out_specs=_H_SPEC,
        out_shape=jax.ShapeDtypeStruct((_NP, _HID), jnp.float32),
    )(degp, s, h, W2, b2)


def kernel(x, edge_index, W1, b1, gamma, beta, running_mean, running_var, W2, b2):
    with jax.enable_x64(False):
        return _kernel32(x, edge_index, W1, b1, gamma, beta, running_mean,
                         running_var, W2, b2)


def _kernel32(x, edge_index, W1, b1, gamma, beta, running_mean, running_var, W2, b2):
    x = x.astype(jnp.float32)
    row = edge_index[0].astype(jnp.int32)
    col = edge_index[1].astype(jnp.int32)
    pad = _NW * _EPW - _E
    rowp = jnp.concatenate([row, jnp.full((pad,), _N, jnp.int32)]).reshape(
        _NW, _CHUNKS, _CH)
    colp = jnp.concatenate([col, jnp.full((pad,), _N, jnp.int32)]).reshape(
        _NW, _CHUNKS, _CH)
    ones16 = jnp.ones((_CH, _DEGW), jnp.float32)
    z16 = jnp.zeros((_NP, _DEGW), jnp.float32)
    zblk = jnp.zeros((_CH, _HH), jnp.float32)

    h = _prep_h(x, W1.astype(jnp.float32), b1, gamma, beta,
                running_mean, running_var)
    degp = _deg_partials(rowp, ones16, z16)
    u = _make_u0(degp, h)
    s = _spmv_partials(u, rowp, colp, zblk)
    u = _combine_u(degp, s, h)
    s = _spmv_partials(u, rowp, colp, zblk)
    out = _final(degp, s, h, W2.astype(jnp.float32), b2)
    return out[:_N]
